# baseline (speedup 1.0000x reference)
import jax
import jax.numpy as jnp
from jax import lax
from jax.experimental import pallas as pl
from jax.experimental.pallas import tpu as pltpu
from jax.experimental.pallas import tpu_sc as plsc

EMB = 128
BATCH = 16384

_INFO = plsc.get_sparse_core_info()
NC = _INFO.num_cores
NS = _INFO.num_subcores
L = _INFO.num_lanes
NW = NC * NS
ROWS_PER_W = BATCH // NW
CHUNK = 128
NCHUNK = ROWS_PER_W // CHUNK
NGRP = CHUNK // L


def _lane_shuffle(x, idx):
    dnums = lax.GatherDimensionNumbers(
        offset_dims=(), collapsed_slice_dims=(0,), start_index_map=(0,))
    return lax.gather(x, idx[:, None], dnums, (1,),
                      mode=lax.GatherScatterMode.PROMISE_IN_BOUNDS)


def _body(u_hbm, v_hbm, user_hbm, item_hbm, out_hbm,
          uidx_v, vidx_v, ue_v, ve_v, out_v, sem0, sem1):
    wid = lax.axis_index("s") * NC + lax.axis_index("c")
    base = wid * ROWS_PER_W

    lanes = lax.iota(jnp.int32, L)
    sems = (sem0, sem1)

    pltpu.sync_copy(u_hbm.at[pl.ds(base, ROWS_PER_W)], uidx_v)
    pltpu.sync_copy(v_hbm.at[pl.ds(base, ROWS_PER_W)], vidx_v)

    def fire(c):
        b = c % 2
        cp_u = pltpu.make_async_copy(
            user_hbm.at[uidx_v.at[pl.ds(c * CHUNK, CHUNK)]], ue_v.at[b],
            sems[b])
        cp_v = pltpu.make_async_copy(
            item_hbm.at[vidx_v.at[pl.ds(c * CHUNK, CHUNK)]], ve_v.at[b],
            sems[b])
        cp_u.start()
        cp_v.start()
        return cp_u, cp_v

    inflight = fire(0)
    for c in range(NCHUNK):
        nxt = fire(c + 1) if c + 1 < NCHUNK else None
        inflight[0].wait()
        inflight[1].wait()
        inflight = nxt
        b = c % 2

        def group_body(g, _):
            ubuf = ue_v.at[b]
            vbuf = ve_v.at[b]
            vecs = []
            for r in range(L):
                urow = ubuf.at[g * L + r]
                vrow = vbuf.at[g * L + r]
                acc = urow[pl.ds(0, L)] * vrow[pl.ds(0, L)]
                for k in range(1, EMB // L):
                    acc = acc + urow[pl.ds(k * L, L)] * vrow[pl.ds(k * L, L)]
                vecs.append(acc)
            d = 1
            while len(vecs) > 1:
                nxt_vecs = []
                m = (lanes & d) != 0
                for i in range(0, len(vecs), 2):
                    a, bb = vecs[i], vecs[i + 1]
                    pa = a + _lane_shuffle(a, lanes ^ d)
                    pb = bb + _lane_shuffle(bb, lanes ^ d)
                    nxt_vecs.append(jnp.where(m, pb, pa))
                vecs = nxt_vecs
                d *= 2
            out_v[pl.ds(c * CHUNK + g * L, L)] = vecs[0]
            return 0

        lax.fori_loop(0, NGRP, group_body, 0)

    pltpu.sync_copy(out_v, out_hbm.at[pl.ds(base, ROWS_PER_W)])


@jax.jit
def kernel(u, v, user_emb, item_emb):
    mesh = plsc.VectorSubcoreMesh(core_axis_name="c", subcore_axis_name="s")
    run = pl.kernel(
        _body,
        mesh=mesh,
        out_type=jax.ShapeDtypeStruct((BATCH,), jnp.float32),
        scratch_types=[
            pltpu.VMEM((ROWS_PER_W,), jnp.int32),
            pltpu.VMEM((ROWS_PER_W,), jnp.int32),
            pltpu.VMEM((2, CHUNK, EMB), jnp.float32),
            pltpu.VMEM((2, CHUNK, EMB), jnp.float32),
            pltpu.VMEM((ROWS_PER_W,), jnp.float32),
            pltpu.SemaphoreType.DMA,
            pltpu.SemaphoreType.DMA,
        ],
    )
    return run(u, v, user_emb, item_emb)


# speedup vs baseline: 1.5984x; 1.5984x over previous
import jax
import jax.numpy as jnp
from jax import lax
from jax.experimental import pallas as pl
from jax.experimental.pallas import tpu as pltpu
from jax.experimental.pallas import tpu_sc as plsc

EMB = 128
BATCH = 16384

_INFO = plsc.get_sparse_core_info()
NC = _INFO.num_cores
NS = _INFO.num_subcores
L = _INFO.num_lanes
NW = NC * NS
ROWS_PER_W = BATCH // NW
CHUNK = 128
NCHUNK = ROWS_PER_W // CHUNK
NGRP = CHUNK // L


def _lane_shuffle(x, idx):
    dnums = lax.GatherDimensionNumbers(
        offset_dims=(), collapsed_slice_dims=(0,), start_index_map=(0,))
    return lax.gather(x, idx[:, None], dnums, (1,),
                      mode=lax.GatherScatterMode.PROMISE_IN_BOUNDS)


def _body(u_hbm, v_hbm, user_hbm, item_hbm, out_hbm,
          uidx_v, vidx_v, ue_v, ve_v, out_v, sem0, sem1):
    wid = lax.axis_index("s") * NC + lax.axis_index("c")
    base = wid * ROWS_PER_W

    lanes = lax.iota(jnp.int32, L)
    sems = (sem0, sem1)

    pltpu.sync_copy(u_hbm.at[pl.ds(base, ROWS_PER_W)], uidx_v)
    pltpu.sync_copy(v_hbm.at[pl.ds(base, ROWS_PER_W)], vidx_v)

    def fire(c):
        b = c % 2
        cp_u = pltpu.make_async_copy(
            user_hbm.at[uidx_v.at[pl.ds(c * CHUNK, CHUNK)]], ue_v.at[b],
            sems[b])
        cp_v = pltpu.make_async_copy(
            item_hbm.at[vidx_v.at[pl.ds(c * CHUNK, CHUNK)]], ve_v.at[b],
            sems[b])
        cp_u.start()
        cp_v.start()
        return cp_u, cp_v

    inflight = fire(0)
    for c in range(NCHUNK):
        nxt = fire(c + 1) if c + 1 < NCHUNK else None
        inflight[0].wait()
        inflight[1].wait()
        inflight = nxt
        b = c % 2

        def group_body(g, _):
            ubuf = ue_v.at[b]
            vbuf = ve_v.at[b]

            def row_body(r, vec):
                urow = ubuf.at[g * L + r]
                vrow = vbuf.at[g * L + r]
                acc = urow[pl.ds(0, L)] * vrow[pl.ds(0, L)]
                for k in range(1, EMB // L):
                    acc = acc + urow[pl.ds(k * L, L)] * vrow[pl.ds(k * L, L)]
                for sh in (8, 4, 2, 1):
                    acc = acc + _lane_shuffle(acc, lanes ^ sh)
                return jnp.where(lanes == r, acc, vec)

            vec = lax.fori_loop(0, L, row_body, jnp.zeros((L,), jnp.float32))
            out_v[pl.ds(c * CHUNK + g * L, L)] = vec
            return 0

        lax.fori_loop(0, NGRP, group_body, 0)

    pltpu.sync_copy(out_v, out_hbm.at[pl.ds(base, ROWS_PER_W)])


@jax.jit
def kernel(u, v, user_emb, item_emb):
    mesh = plsc.VectorSubcoreMesh(core_axis_name="c", subcore_axis_name="s")
    run = pl.kernel(
        _body,
        mesh=mesh,
        out_type=jax.ShapeDtypeStruct((BATCH,), jnp.float32),
        scratch_types=[
            pltpu.VMEM((ROWS_PER_W,), jnp.int32),
            pltpu.VMEM((ROWS_PER_W,), jnp.int32),
            pltpu.VMEM((2, CHUNK, EMB), jnp.float32),
            pltpu.VMEM((2, CHUNK, EMB), jnp.float32),
            pltpu.VMEM((ROWS_PER_W,), jnp.float32),
            pltpu.SemaphoreType.DMA,
            pltpu.SemaphoreType.DMA,
        ],
    )
    return run(u, v, user_emb, item_emb)
